# Initial kernel scaffold; baseline (speedup 1.0000x reference)
import jax
import jax.numpy as jnp
from jax import lax
from jax.experimental import pallas as pl
from jax.experimental.pallas import tpu as pltpu
from jax.experimental.pallas import tpu_sc as plsc

U = 5000
I = 5000
N = U + I
D = 128
E = 320000
NC = 2
NS = 16
EPT = E // (NC * NS)
B = 80
NCH = EPT // B
RPT = N // NS
ZR = 125
F32 = jnp.float32


def _spmm_body(cur, cols, rows, vals, out, colbuf, rowbuf, valbuf, gath,
               zbuf, zacc, sem):
    c = lax.axis_index("c")
    s = lax.axis_index("s")

    @pl.loop(0, ZR)
    def _zero_zbuf(r):
        for j in range(D // 16):
            zbuf[r, pl.ds(j * 16, 16)] = jnp.zeros((16,), F32)

    for t in range(RPT // ZR):
        pltpu.sync_copy(zbuf, zacc.at[pl.ds(s * RPT + t * ZR, ZR)])
    plsc.subcore_barrier()

    base0 = c * (NS * EPT) + s * EPT

    @pl.loop(0, NCH)
    def _chunk(i):
        base = base0 + i * B
        pltpu.sync_copy(cols.at[pl.ds(base, B)], colbuf)
        pltpu.sync_copy(rows.at[pl.ds(base, B)], rowbuf)
        pltpu.sync_copy(vals.at[pl.ds(base, B)], valbuf)
        pltpu.async_copy(cur.at[colbuf], gath, sem).wait()

        @pl.loop(0, B)
        def _edge(e):
            v = valbuf[e]
            for j in range(D // 16):
                sl = pl.ds(j * 16, 16)
                gath[e, sl] = gath[e, sl] * v

        pltpu.sync_copy(gath, zacc.at[rowbuf], add=True)

    plsc.subcore_barrier()
    pltpu.sync_copy(zacc.at[pl.ds(s * RPT, RPT)],
                    out.at[pl.ds(c * N + s * RPT, RPT)])


_spmm = pl.kernel(
    _spmm_body,
    out_type=jax.ShapeDtypeStruct((NC * N, D), F32),
    mesh=plsc.VectorSubcoreMesh(core_axis_name="c", subcore_axis_name="s"),
    scratch_types=[
        pltpu.VMEM((B,), jnp.int32),
        pltpu.VMEM((B,), jnp.int32),
        pltpu.VMEM((B,), F32),
        pltpu.VMEM((B, D), F32),
        pltpu.VMEM((ZR, D), F32),
        pltpu.VMEM_SHARED((N, D), F32),
        pltpu.SemaphoreType.DMA,
    ],
)


def _dense0_body(ego_ref, uw_ref, iw_ref, zp_ref,
                 hu_ref, hi_ref, z_ref, g_ref, ego1_ref):
    ego = ego_ref[...]
    eu = ego[:U]
    ei = ego[U:]
    hu = jnp.dot(eu, uw_ref[...], preferred_element_type=F32)
    hi = jnp.dot(ei, iw_ref[...], preferred_element_type=F32)
    z = zp_ref[:N] + zp_ref[N:]
    lam_u = lax.dot_general(hu, eu, (((0,), (0,)), ((), ())),
                            preferred_element_type=F32)
    lam_i = lax.dot_general(hi, ei, (((0,), (0,)), ((), ())),
                            preferred_element_type=F32)
    g = jnp.concatenate(
        [jnp.dot(hu, lam_u, preferred_element_type=F32),
         jnp.dot(hi, lam_i, preferred_element_type=F32)], axis=0)
    hu_ref[...] = hu
    hi_ref[...] = hi
    z_ref[...] = z
    g_ref[...] = g
    ego1_ref[...] = (z + g) * 0.5


_dense0 = pl.pallas_call(
    _dense0_body,
    out_shape=(
        jax.ShapeDtypeStruct((U, D), F32),
        jax.ShapeDtypeStruct((I, D), F32),
        jax.ShapeDtypeStruct((N, D), F32),
        jax.ShapeDtypeStruct((N, D), F32),
        jax.ShapeDtypeStruct((N, D), F32),
    ),
)


def _dense1_body(ego0_ref, ego1_ref, hu_ref, hi_ref, zp_ref,
                 z_ref, g_ref, fu_ref, fi_ref):
    ego1 = ego1_ref[...]
    eu = ego1[:U]
    ei = ego1[U:]
    hu = hu_ref[...]
    hi = hi_ref[...]
    z = zp_ref[:N] + zp_ref[N:]
    lam_u = lax.dot_general(hu, eu, (((0,), (0,)), ((), ())),
                            preferred_element_type=F32)
    lam_i = lax.dot_general(hi, ei, (((0,), (0,)), ((), ())),
                            preferred_element_type=F32)
    g = jnp.concatenate(
        [jnp.dot(hu, lam_u, preferred_element_type=F32),
         jnp.dot(hi, lam_i, preferred_element_type=F32)], axis=0)
    ego2 = (z + g) * 0.5
    final = (ego0_ref[...] + ego1 + ego2) * (1.0 / 3.0)
    z_ref[...] = z
    g_ref[...] = g
    fu_ref[...] = final[:U]
    fi_ref[...] = final[U:]


_dense1 = pl.pallas_call(
    _dense1_body,
    out_shape=(
        jax.ShapeDtypeStruct((N, D), F32),
        jax.ShapeDtypeStruct((N, D), F32),
        jax.ShapeDtypeStruct((U, D), F32),
        jax.ShapeDtypeStruct((I, D), F32),
    ),
)


def kernel(user_emb, item_emb, user_hyper_emb, item_hyper_emb,
           adj_indices, adj_values):
    rows = adj_indices[0]
    cols = adj_indices[1]
    ego0 = jnp.concatenate([user_emb, item_emb], axis=0)

    zp0 = _spmm(ego0, cols, rows, adj_values)
    hu, hi, z0, g0, ego1 = _dense0(ego0, user_hyper_emb, item_hyper_emb, zp0)
    zp1 = _spmm(ego1, cols, rows, adj_values)
    z1, g1, fu, fi = _dense1(ego0, ego1, hu, hi, zp1)

    return (fu, fi, (z0, z1), (g0, g1))


# speedup vs baseline: 4.1009x; 4.1009x over previous
import functools

import jax
import jax.numpy as jnp
from jax import lax
from jax.experimental import pallas as pl
from jax.experimental.pallas import tpu as pltpu
from jax.experimental.pallas import tpu_sc as plsc

U = 5000
I = 5000
N = U + I
D = 128
E = 320000
NC = 2
NS = 16
EPT = E // (NC * NS)
B = 80
NCH = EPT // B
NP = 10240
RPT = NP // NS
ZR = 128
F32 = jnp.float32


def _spmm_body(cur, cols, rows, vals, out, colbuf, rowbuf, valbuf, gath,
               zbuf, zacc, sem):
    c = lax.axis_index("c")
    s = lax.axis_index("s")

    @pl.loop(0, ZR)
    def _zero_zbuf(r):
        for j in range(D // 16):
            zbuf[r, pl.ds(j * 16, 16)] = jnp.zeros((16,), F32)

    for t in range(RPT // ZR):
        pltpu.sync_copy(zbuf, zacc.at[pl.ds(s * RPT + t * ZR, ZR)])
    plsc.subcore_barrier()

    base0 = c * (NS * EPT) + s * EPT

    @pl.loop(0, NCH)
    def _chunk(i):
        base = base0 + i * B
        pltpu.sync_copy(cols.at[pl.ds(base, B)], colbuf)
        pltpu.sync_copy(rows.at[pl.ds(base, B)], rowbuf)
        pltpu.sync_copy(vals.at[pl.ds(base, B)], valbuf)
        pltpu.async_copy(cur.at[colbuf], gath, sem).wait()

        @pl.loop(0, B // 16)
        def _grp(g):
            vvec = valbuf[pl.ds(g * 16, 16)]
            for i in range(16):
                v = vvec[i]
                e = g * 16 + i
                for j in range(D // 16):
                    sl = pl.ds(j * 16, 16)
                    gath[e, sl] = gath[e, sl] * v

        pltpu.sync_copy(gath, zacc.at[rowbuf], add=True)

    plsc.subcore_barrier()
    pltpu.sync_copy(zacc.at[pl.ds(s * RPT, RPT)],
                    out.at[pl.ds(c * NP + s * RPT, RPT)])


@functools.cache
def _get_spmm():
    return pl.kernel(
        _spmm_body,
        out_type=jax.ShapeDtypeStruct((NC * NP, D), F32),
        mesh=plsc.VectorSubcoreMesh(core_axis_name="c", subcore_axis_name="s",
                                    num_cores=NC, num_subcores=NS),
        scratch_types=[
            pltpu.VMEM((B,), jnp.int32),
            pltpu.VMEM((B,), jnp.int32),
            pltpu.VMEM((B,), F32),
            pltpu.VMEM((B, D), F32),
            pltpu.VMEM((ZR, D), F32),
            pltpu.VMEM_SHARED((NP, D), F32),
            pltpu.SemaphoreType.DMA,
        ],
    )


def _dense0_body(ego_ref, uw_ref, iw_ref, zp_ref,
                 hu_ref, hi_ref, z_ref, g_ref, ego1_ref):
    ego = ego_ref[...]
    eu = ego[:U]
    ei = ego[U:]
    hu = jnp.dot(eu, uw_ref[...], preferred_element_type=F32)
    hi = jnp.dot(ei, iw_ref[...], preferred_element_type=F32)
    z = zp_ref[:N] + zp_ref[NP:NP + N]
    lam_u = lax.dot_general(hu, eu, (((0,), (0,)), ((), ())),
                            preferred_element_type=F32)
    lam_i = lax.dot_general(hi, ei, (((0,), (0,)), ((), ())),
                            preferred_element_type=F32)
    g = jnp.concatenate(
        [jnp.dot(hu, lam_u, preferred_element_type=F32),
         jnp.dot(hi, lam_i, preferred_element_type=F32)], axis=0)
    hu_ref[...] = hu
    hi_ref[...] = hi
    z_ref[...] = z
    g_ref[...] = g
    ego1_ref[...] = (z + g) * 0.5


_dense0 = pl.pallas_call(
    _dense0_body,
    out_shape=(
        jax.ShapeDtypeStruct((U, D), F32),
        jax.ShapeDtypeStruct((I, D), F32),
        jax.ShapeDtypeStruct((N, D), F32),
        jax.ShapeDtypeStruct((N, D), F32),
        jax.ShapeDtypeStruct((N, D), F32),
    ),
)


def _dense1_body(ego0_ref, ego1_ref, hu_ref, hi_ref, zp_ref,
                 z_ref, g_ref, fu_ref, fi_ref):
    ego1 = ego1_ref[...]
    eu = ego1[:U]
    ei = ego1[U:]
    hu = hu_ref[...]
    hi = hi_ref[...]
    z = zp_ref[:N] + zp_ref[NP:NP + N]
    lam_u = lax.dot_general(hu, eu, (((0,), (0,)), ((), ())),
                            preferred_element_type=F32)
    lam_i = lax.dot_general(hi, ei, (((0,), (0,)), ((), ())),
                            preferred_element_type=F32)
    g = jnp.concatenate(
        [jnp.dot(hu, lam_u, preferred_element_type=F32),
         jnp.dot(hi, lam_i, preferred_element_type=F32)], axis=0)
    ego2 = (z + g) * 0.5
    final = (ego0_ref[...] + ego1 + ego2) * (1.0 / 3.0)
    z_ref[...] = z
    g_ref[...] = g
    fu_ref[...] = final[:U]
    fi_ref[...] = final[U:]


_dense1 = pl.pallas_call(
    _dense1_body,
    out_shape=(
        jax.ShapeDtypeStruct((N, D), F32),
        jax.ShapeDtypeStruct((N, D), F32),
        jax.ShapeDtypeStruct((U, D), F32),
        jax.ShapeDtypeStruct((I, D), F32),
    ),
)


def kernel(user_emb, item_emb, user_hyper_emb, item_hyper_emb,
           adj_indices, adj_values):
    rows = adj_indices[0]
    cols = adj_indices[1]
    ego0 = jnp.concatenate([user_emb, item_emb], axis=0)

    spmm = _get_spmm()
    zp0 = spmm(ego0, cols, rows, adj_values)
    hu, hi, z0, g0, ego1 = _dense0(ego0, user_hyper_emb, item_hyper_emb, zp0)
    zp1 = spmm(ego1, cols, rows, adj_values)
    z1, g1, fu, fi = _dense1(ego0, ego1, hu, hi, zp1)

    return (fu, fi, (z0, z1), (g0, g1))
